# baseline (speedup 1.0000x reference)
import jax
import jax.numpy as jnp
from jax.experimental import pallas as pl
from jax.experimental.pallas import tpu as pltpu

_C = 256
_M = 32
_BB = 2
_VMEM = 56 * 1024 * 1024


def _se_block_kernel(x_ref, w1t_ref, b1_ref, w2t_ref, b2_ref, o_ref, *, inv_l):
    x = x_ref[...]
    pooled = jnp.sum(x, axis=2) * inv_l
    h = jnp.dot(pooled, w1t_ref[...],
                precision=jax.lax.Precision.HIGHEST,
                preferred_element_type=jnp.float32)
    h = jnp.maximum(h + b1_ref[...], 0.0)
    s = jnp.dot(h, w2t_ref[...],
                precision=jax.lax.Precision.HIGHEST,
                preferred_element_type=jnp.float32)
    s = jax.nn.sigmoid(s + b2_ref[...])
    o_ref[...] = x * s[:, :, None]


def kernel(x, w1, b1, w2, b2):
    B, C, L = x.shape
    bb = _BB if B % _BB == 0 else 1
    w1t = w1[:, :, 0].T.astype(jnp.float32)
    w2t = w2[:, :, 0].T.astype(jnp.float32)
    b1r = b1.astype(jnp.float32).reshape(1, _M)
    b2r = b2.astype(jnp.float32).reshape(1, _C)

    import functools
    return pl.pallas_call(
        functools.partial(_se_block_kernel, inv_l=1.0 / L),
        out_shape=jax.ShapeDtypeStruct((B, C, L), x.dtype),
        grid=(B // bb,),
        in_specs=[
            pl.BlockSpec((bb, C, L), lambda i: (i, 0, 0)),
            pl.BlockSpec((C, _M), lambda i: (0, 0)),
            pl.BlockSpec((1, _M), lambda i: (0, 0)),
            pl.BlockSpec((_M, C), lambda i: (0, 0)),
            pl.BlockSpec((1, _C), lambda i: (0, 0)),
        ],
        out_specs=pl.BlockSpec((bb, C, L), lambda i: (i, 0, 0)),
        compiler_params=pltpu.CompilerParams(
            dimension_semantics=("parallel",),
            vmem_limit_bytes=_VMEM),
    )(x, w1t, b1r, w2t, b2r)


# speedup vs baseline: 1.0020x; 1.0020x over previous
import functools

import jax
import jax.numpy as jnp
from jax.experimental import pallas as pl
from jax.experimental.pallas import tpu as pltpu

_C = 256
_M = 32
_BB = 4
_VMEM = 56 * 1024 * 1024


def _se_block_kernel(x_ref, p_ref, o_ref, *, inv_l):
    x = x_ref[...]
    pooled = jnp.sum(x, axis=2) * inv_l
    w1m = p_ref[0:_M, :]
    w2t = p_ref[_M:2 * _M, :]
    b1 = p_ref[2 * _M:2 * _M + 1, 0:_M]
    b2 = p_ref[2 * _M + 1:2 * _M + 2, :]
    h = jax.lax.dot_general(
        pooled, w1m, (((1,), (1,)), ((), ())),
        precision=jax.lax.Precision.HIGHEST,
        preferred_element_type=jnp.float32)
    h = jnp.maximum(h + b1, 0.0)
    s = jnp.dot(h, w2t,
                precision=jax.lax.Precision.HIGHEST,
                preferred_element_type=jnp.float32)
    s = jax.nn.sigmoid(s + b2)
    o_ref[...] = x * s[:, :, None]


def kernel(x, w1, b1, w2, b2):
    B, C, L = x.shape
    bb = _BB if B % _BB == 0 else 1
    w1m = w1[:, :, 0].astype(jnp.float32)
    w2t = w2[:, :, 0].T.astype(jnp.float32)
    b1p = jnp.zeros((1, _C), jnp.float32).at[0, :_M].set(b1.astype(jnp.float32))
    b2p = b2.astype(jnp.float32).reshape(1, _C)
    packed = jnp.concatenate(
        [w1m, w2t, b1p, b2p,
         jnp.zeros((2 * _M + 8 - (2 * _M + 2), _C), jnp.float32)], axis=0)

    return pl.pallas_call(
        functools.partial(_se_block_kernel, inv_l=1.0 / L),
        out_shape=jax.ShapeDtypeStruct((B, C, L), x.dtype),
        grid=(B // bb,),
        in_specs=[
            pl.BlockSpec((bb, C, L), lambda i: (i, 0, 0)),
            pl.BlockSpec(packed.shape, lambda i: (0, 0)),
        ],
        out_specs=pl.BlockSpec((bb, C, L), lambda i: (i, 0, 0)),
        compiler_params=pltpu.CompilerParams(
            dimension_semantics=("parallel",),
            vmem_limit_bytes=_VMEM),
    )(x, packed)


# speedup vs baseline: 1.0331x; 1.0311x over previous
import functools

import jax
import jax.numpy as jnp
from jax.experimental import pallas as pl
from jax.experimental.pallas import tpu as pltpu

_C = 256
_M = 32
_BB = 4
_VMEM = 56 * 1024 * 1024


def _se_block_kernel(x_ref, w1_ref, b1_ref, w2_ref, b2_ref, o_ref, *, inv_l):
    x = x_ref[...]
    pooled = jnp.sum(x, axis=2) * inv_l
    h = jax.lax.dot_general(
        pooled, w1_ref[...], (((1,), (1,)), ((), ())),
        precision=jax.lax.Precision.HIGHEST,
        preferred_element_type=jnp.float32)
    h = jnp.maximum(h + b1_ref[...], 0.0)
    s = jax.lax.dot_general(
        h, w2_ref[...], (((1,), (1,)), ((), ())),
        precision=jax.lax.Precision.HIGHEST,
        preferred_element_type=jnp.float32)
    s = jax.nn.sigmoid(s + b2_ref[...])
    o_ref[...] = x * s[:, :, None]


def kernel(x, w1, b1, w2, b2):
    B, C, L = x.shape
    bb = _BB if B % _BB == 0 else 1
    w1m = w1.reshape(_M, C)
    w2m = w2.reshape(C, _M)
    b1r = b1.reshape(1, _M)
    b2r = b2.reshape(1, _C)

    return pl.pallas_call(
        functools.partial(_se_block_kernel, inv_l=1.0 / L),
        out_shape=jax.ShapeDtypeStruct((B, C, L), x.dtype),
        grid=(B // bb,),
        in_specs=[
            pl.BlockSpec((bb, C, L), lambda i: (i, 0, 0)),
            pl.BlockSpec((_M, C), lambda i: (0, 0)),
            pl.BlockSpec((1, _M), lambda i: (0, 0)),
            pl.BlockSpec((C, _M), lambda i: (0, 0)),
            pl.BlockSpec((1, _C), lambda i: (0, 0)),
        ],
        out_specs=pl.BlockSpec((bb, C, L), lambda i: (i, 0, 0)),
        compiler_params=pltpu.CompilerParams(
            dimension_semantics=("parallel",),
            vmem_limit_bytes=_VMEM),
    )(x, w1m, b1r, w2m, b2r)
